# Initial kernel scaffold; baseline (speedup 1.0000x reference)
import jax
import jax.numpy as jnp
from jax.experimental import pallas as pl


def kernel(x, edge_index, adj, W1, a1s, a1d, b1, W2, a2s, a2d, b2, W3, a3s, a3d, b3, Wm1, bm1, Wm2, bm2, Wm3, bm3):
    raise NotImplementedError("write your pallas kernel here")


# speedup vs baseline: 32.1857x; 32.1857x over previous
import functools

import jax
import jax.numpy as jnp
from jax import lax
from jax.experimental import pallas as pl
from jax.experimental.pallas import tpu as pltpu
from jax.experimental.pallas import tpu_sc as plsc

H2 = 64
NC, NS = 2, 16
NW = NC * NS
KB = 128
NSEG_PAD = 10240
RPT = NSEG_PAD // NS


def _make_gat_sc(nh, n_nodes, nb):
    D = 128
    CH = 8 // nh
    mesh = plsc.VectorSubcoreMesh(core_axis_name="c", subcore_axis_name="s")
    out_type = [jax.ShapeDtypeStruct((NC, NSEG_PAD, D), jnp.float32)] + [
        jax.ShapeDtypeStruct((NC, NSEG_PAD), jnp.float32) for _ in range(nh)
    ]
    scratch = (
        [
            pltpu.VMEM((nb, KB), jnp.int32),
            pltpu.VMEM((nb, KB), jnp.int32),
        ]
        + [pltpu.VMEM((KB,), jnp.int32) for _ in range(2 * nh)]
        + [pltpu.VMEM((KB,), jnp.float32) for _ in range(2 * nh)]
        + [pltpu.VMEM((KB,), jnp.float32) for _ in range(nh)]
        + [
            pltpu.VMEM((KB, D), jnp.float32),
            pltpu.VMEM((RPT,), jnp.float32),
            pltpu.VMEM_SHARED((NSEG_PAD, D), jnp.float32),
        ]
        + [pltpu.VMEM_SHARED((NSEG_PAD,), jnp.float32) for _ in range(nh)]
        + [pltpu.SemaphoreType.DMA]
    )

    @functools.partial(
        pl.kernel, out_type=out_type, mesh=mesh, scratch_types=scratch,
        compiler_params=pltpu.CompilerParams(needs_layout_passes=False))
    def gat(h_hbm, src_hbm, dst_hbm, sc_hbm, rows_out, *rest):
        den_outs = rest[:nh]
        r2 = rest[nh:]
        src_v, dst_v = r2[0], r2[1]
        gidx = r2[2:2 + 2 * nh]
        sbufs = r2[2 + 2 * nh:2 + 4 * nh]
        ex_bufs = r2[2 + 4 * nh:2 + 5 * nh]
        rows_v, den_b, acc_s = r2[2 + 5 * nh:5 + 5 * nh]
        den_sh = r2[5 + 5 * nh:5 + 6 * nh]
        sem = r2[-1]
        stride = 2 * nh

        c = lax.axis_index("c")
        s = lax.axis_index("s")
        w = c * NS + s
        zero16 = jnp.zeros((16,), jnp.float32)

        pltpu.sync_copy(src_hbm.at[w], src_v)
        pltpu.sync_copy(dst_hbm.at[w], dst_v)

        @pl.loop(0, KB)
        def _zr(r):
            for cc in range(8):
                rows_v[r, pl.ds(cc * 16, 16)] = zero16

        @pl.loop(0, RPT // 16)
        def _zd(i):
            den_b[pl.ds(i * 16, 16)] = zero16

        base = s * RPT
        for i in range(RPT // KB):
            pltpu.sync_copy(rows_v, acc_s.at[pl.ds(base + i * KB, KB)])
        for hd in range(nh):
            pltpu.sync_copy(den_b, den_sh[hd].at[pl.ds(base, RPT)])
        plsc.subcore_barrier()

        @pl.loop(0, nb)
        def _batch(b):
            for i in range(KB // 16):
                sv = src_v[b, pl.ds(i * 16, 16)] * stride
                dv = dst_v[b, pl.ds(i * 16, 16)] * stride
                for hd in range(nh):
                    gidx[2 * hd][pl.ds(i * 16, 16)] = sv + (2 * hd)
                    gidx[2 * hd + 1][pl.ds(i * 16, 16)] = dv + (2 * hd + 1)
            cps = [pltpu.async_copy(sc_hbm.at[gidx[k]], sbufs[k], sem)
                   for k in range(2 * nh)]
            cps.append(pltpu.async_copy(h_hbm.at[src_v.at[b]], rows_v, sem))
            for cp in cps:
                cp.wait()
            for i in range(KB // 16):
                sl = pl.ds(i * 16, 16)
                for hd in range(nh):
                    e = sbufs[2 * hd][sl] + sbufs[2 * hd + 1][sl]
                    e = jnp.where(e > 0, e, 0.2 * e)
                    ex_bufs[hd][sl] = jnp.exp(e)
            for hd in range(nh):
                pltpu.sync_copy(ex_bufs[hd], den_sh[hd].at[dst_v.at[b]],
                                add=True)

            @pl.loop(0, KB)
            def _scale(j):
                jv = jnp.zeros((16,), jnp.int32) + j
                for hd in range(nh):
                    exj = plsc.load_gather(ex_bufs[hd], [jv])
                    for cc in range(hd * CH, (hd + 1) * CH):
                        rows_v[j, pl.ds(cc * 16, 16)] = (
                            rows_v[j, pl.ds(cc * 16, 16)] * exj)

            pltpu.sync_copy(rows_v, acc_s.at[dst_v.at[b]], add=True)

        plsc.subcore_barrier()

        for i in range(RPT // KB):
            pltpu.sync_copy(acc_s.at[pl.ds(base + i * KB, KB)], rows_v)
            pltpu.sync_copy(rows_v, rows_out.at[c, pl.ds(base + i * KB, KB)])
        for hd in range(nh):
            pltpu.sync_copy(den_sh[hd].at[pl.ds(base, RPT)], den_b)
            pltpu.sync_copy(den_b, den_outs[hd].at[c, pl.ds(base, RPT)])

    return gat


def _proj1_body(x_ref, w1_ref, wm1_ref, a1_ref, h_ref, xw_ref, sc_ref):
    xb = x_ref[...]
    h = jnp.dot(xb, w1_ref[...], preferred_element_type=jnp.float32)
    h_ref[...] = h
    xw_ref[...] = jnp.dot(xb, wm1_ref[...], preferred_element_type=jnp.float32)
    sc_ref[...] = jnp.dot(h, a1_ref[...], preferred_element_type=jnp.float32)


def _gcn1_body(adj_ref, xw_ref, wm23_ref, bm1_ref, xlm_ref, t23_ref):
    acc = jnp.dot(adj_ref[...], xw_ref[...], preferred_element_type=jnp.float32)
    xlm = jnp.maximum(acc + bm1_ref[...], 0.0)
    xlm_ref[...] = xlm
    t23_ref[...] = jnp.dot(xlm, wm23_ref[...],
                           preferred_element_type=jnp.float32)


def _gcn2_body(adj_ref, t23_ref, bm2_ref, bm3_ref, mu_ref, lv_ref):
    acc = jnp.dot(adj_ref[...], t23_ref[...], preferred_element_type=jnp.float32)
    mu_ref[...] = acc[:, :H2] + bm2_ref[...]
    lv_ref[...] = acc[:, H2:] + bm3_ref[...]


def _fin1_body(rp_ref, dp_ref, b1_ref, w23_ref, a23_ref,
               xl_ref, h23_ref, sc_ref):
    rp = rp_ref[...]
    dp = dp_ref[...]
    den = dp[0, 0] + dp[0, 1]
    xl = jnp.maximum((rp[0] + rp[1]) / (den[:, None] + 1e-16) + b1_ref[...],
                     0.0)
    xl_ref[...] = xl
    h23 = jnp.dot(xl, w23_ref[...], preferred_element_type=jnp.float32)
    h23_ref[...] = h23
    sc_ref[...] = jnp.dot(h23, a23_ref[...], preferred_element_type=jnp.float32)


def _fin23_body(rp_ref, dA_ref, dB_ref, b2_ref, b3_ref, mu_ref, lv_ref):
    rp = rp_ref[...]
    r = rp[0] + rp[1]
    dA = dA_ref[...]
    dB = dB_ref[...]
    denA = dA[0, 0] + dA[0, 1]
    denB = dB[0, 0] + dB[0, 1]
    mu_ref[...] = r[:, :H2] / (denA[:, None] + 1e-16) + b2_ref[...]
    lv_ref[...] = r[:, H2:] / (denB[:, None] + 1e-16) + b3_ref[...]


def _full(shape):
    return pl.BlockSpec(shape, lambda i: tuple(0 for _ in shape))


def kernel(x, edge_index, adj, W1, a1s, a1d, b1, W2, a2s, a2d, b2,
           W3, a3s, a3d, b3, Wm1, bm1, Wm2, bm2, Wm3, bm3):
    n = x.shape[0]
    d_in = x.shape[1]
    h1d = W1.shape[1]
    e = edge_index.shape[1]
    f32 = jnp.float32

    loops = jnp.arange(n, dtype=edge_index.dtype)
    src = jnp.concatenate([edge_index[0], loops])
    dst = jnp.concatenate([edge_index[1], loops])
    ne = e + n
    nb = -(-ne // (NW * KB))
    pad = NW * nb * KB - ne
    src = jnp.concatenate([src, jnp.zeros((pad,), src.dtype)])
    dst = jnp.concatenate([dst, jnp.full((pad,), n, dst.dtype)])
    src = src.reshape(NW, nb, KB)
    dst = dst.reshape(NW, nb, KB)

    A1 = jnp.stack([a1s, a1d], axis=1)
    z = jnp.zeros_like(a2s)
    A23 = jnp.stack([jnp.concatenate([a2s, z]),
                     jnp.concatenate([a2d, z]),
                     jnp.concatenate([z, a3s]),
                     jnp.concatenate([z, a3d])], axis=1)
    W23 = jnp.concatenate([W2, W3], axis=1)
    Wm23 = jnp.concatenate([Wm2, Wm3], axis=1)
    b1r = b1.reshape(1, h1d)
    bm1r = bm1.reshape(1, h1d)
    b2r = b2.reshape(1, H2)
    b3r = b3.reshape(1, H2)
    bm2r = bm2.reshape(1, H2)
    bm3r = bm3.reshape(1, H2)

    BN = 1000
    gn = n // BN
    BR = 200
    gr = n // BR

    h1, xw, sc1 = pl.pallas_call(
        _proj1_body,
        grid=(gn,),
        in_specs=[pl.BlockSpec((BN, d_in), lambda i: (i, 0)),
                  _full((d_in, h1d)), _full((d_in, h1d)), _full((d_in, 2))],
        out_specs=[pl.BlockSpec((BN, h1d), lambda i: (i, 0)),
                   pl.BlockSpec((BN, h1d), lambda i: (i, 0)),
                   pl.BlockSpec((BN, 2), lambda i: (i, 0))],
        out_shape=[jax.ShapeDtypeStruct((n, h1d), f32),
                   jax.ShapeDtypeStruct((n, h1d), f32),
                   jax.ShapeDtypeStruct((n, 2), f32)],
    )(x, W1, Wm1, A1)

    sc1p = jnp.pad(sc1, ((0, NSEG_PAD - n), (0, 0))).reshape(-1)
    rows1, den1 = _make_gat_sc(1, n, nb)(h1, src, dst, sc1p)

    xlm, t23 = pl.pallas_call(
        _gcn1_body,
        grid=(gr,),
        in_specs=[pl.BlockSpec((BR, n), lambda i: (i, 0)),
                  _full((n, h1d)), _full((h1d, h1d)), _full((1, h1d))],
        out_specs=[pl.BlockSpec((BR, h1d), lambda i: (i, 0)),
                   pl.BlockSpec((BR, h1d), lambda i: (i, 0))],
        out_shape=[jax.ShapeDtypeStruct((n, h1d), f32),
                   jax.ShapeDtypeStruct((n, h1d), f32)],
    )(adj, xw, Wm23, bm1r)

    den1r = den1[:, :n].reshape(NC, gn, BN).transpose(1, 0, 2)
    xl, h23, sc23 = pl.pallas_call(
        _fin1_body,
        grid=(gn,),
        in_specs=[pl.BlockSpec((NC, BN, h1d), lambda i: (0, i, 0)),
                  pl.BlockSpec((1, NC, BN), lambda i: (i, 0, 0)),
                  _full((1, h1d)), _full((h1d, h1d)), _full((h1d, 4))],
        out_specs=[pl.BlockSpec((BN, h1d), lambda i: (i, 0)),
                   pl.BlockSpec((BN, h1d), lambda i: (i, 0)),
                   pl.BlockSpec((BN, 4), lambda i: (i, 0))],
        out_shape=[jax.ShapeDtypeStruct((n, h1d), f32),
                   jax.ShapeDtypeStruct((n, h1d), f32),
                   jax.ShapeDtypeStruct((n, 4), f32)],
    )(rows1, den1r, b1r, W23, A23)

    sc23p = jnp.pad(sc23, ((0, NSEG_PAD - n), (0, 0))).reshape(-1)
    rows23, den2p, den3p = _make_gat_sc(2, n, nb)(h23, src, dst, sc23p)

    mu_m, logvar_m = pl.pallas_call(
        _gcn2_body,
        grid=(gr,),
        in_specs=[pl.BlockSpec((BR, n), lambda i: (i, 0)),
                  _full((n, h1d)), _full((1, H2)), _full((1, H2))],
        out_specs=[pl.BlockSpec((BR, H2), lambda i: (i, 0)),
                   pl.BlockSpec((BR, H2), lambda i: (i, 0))],
        out_shape=[jax.ShapeDtypeStruct((n, H2), f32),
                   jax.ShapeDtypeStruct((n, H2), f32)],
    )(adj, t23, bm2r, bm3r)

    den2r = den2p[:, :n].reshape(NC, gn, BN).transpose(1, 0, 2)
    den3r = den3p[:, :n].reshape(NC, gn, BN).transpose(1, 0, 2)
    mu, logvar = pl.pallas_call(
        _fin23_body,
        grid=(gn,),
        in_specs=[pl.BlockSpec((NC, BN, h1d), lambda i: (0, i, 0)),
                  pl.BlockSpec((1, NC, BN), lambda i: (i, 0, 0)),
                  pl.BlockSpec((1, NC, BN), lambda i: (i, 0, 0)),
                  _full((1, H2)), _full((1, H2))],
        out_specs=[pl.BlockSpec((BN, H2), lambda i: (i, 0)),
                   pl.BlockSpec((BN, H2), lambda i: (i, 0))],
        out_shape=[jax.ShapeDtypeStruct((n, H2), f32),
                   jax.ShapeDtypeStruct((n, H2), f32)],
    )(rows23, den2r, den3r, b2r, b3r)

    return (xl, mu, mu, logvar, xlm, mu_m, mu_m, logvar_m)


# speedup vs baseline: 32.6258x; 1.0137x over previous
import functools

import jax
import jax.numpy as jnp
from jax import lax
from jax.experimental import pallas as pl
from jax.experimental.pallas import tpu as pltpu
from jax.experimental.pallas import tpu_sc as plsc

H2 = 64
NC, NS = 2, 16
NW = NC * NS
KB = 128
NSEG_PAD = 10240
RPT = NSEG_PAD // NS


def _make_gat_sc(nh, n_nodes, nb):
    D = 128
    CH = 8 // nh
    mesh = plsc.VectorSubcoreMesh(core_axis_name="c", subcore_axis_name="s")
    out_type = [jax.ShapeDtypeStruct((NC, NSEG_PAD, D), jnp.float32)] + [
        jax.ShapeDtypeStruct((NC, NSEG_PAD), jnp.float32) for _ in range(nh)
    ]
    scratch = (
        [
            pltpu.VMEM((nb, KB), jnp.int32),
            pltpu.VMEM((nb, KB), jnp.int32),
        ]
        + [pltpu.VMEM((KB,), jnp.int32) for _ in range(2 * nh)]
        + [pltpu.VMEM((KB,), jnp.float32) for _ in range(2 * nh)]
        + [pltpu.VMEM((KB,), jnp.float32) for _ in range(nh)]
        + [
            pltpu.VMEM((KB, D), jnp.float32),
            pltpu.VMEM((RPT,), jnp.float32),
            pltpu.VMEM_SHARED((NSEG_PAD, D), jnp.float32),
        ]
        + [pltpu.VMEM_SHARED((NSEG_PAD,), jnp.float32) for _ in range(nh)]
        + [pltpu.SemaphoreType.DMA]
    )

    @functools.partial(
        pl.kernel, out_type=out_type, mesh=mesh, scratch_types=scratch,
        compiler_params=pltpu.CompilerParams(needs_layout_passes=False))
    def gat(h_hbm, src_hbm, dst_hbm, sc_hbm, rows_out, *rest):
        den_outs = rest[:nh]
        r2 = rest[nh:]
        src_v, dst_v = r2[0], r2[1]
        gidx = r2[2:2 + 2 * nh]
        sbufs = r2[2 + 2 * nh:2 + 4 * nh]
        ex_bufs = r2[2 + 4 * nh:2 + 5 * nh]
        rows_v, den_b, acc_s = r2[2 + 5 * nh:5 + 5 * nh]
        den_sh = r2[5 + 5 * nh:5 + 6 * nh]
        sem = r2[-1]
        stride = 2 * nh

        c = lax.axis_index("c")
        s = lax.axis_index("s")
        w = c * NS + s
        zero16 = jnp.zeros((16,), jnp.float32)

        pltpu.sync_copy(src_hbm.at[w], src_v)
        pltpu.sync_copy(dst_hbm.at[w], dst_v)

        @pl.loop(0, KB)
        def _zr(r):
            for cc in range(8):
                rows_v[r, pl.ds(cc * 16, 16)] = zero16

        @pl.loop(0, RPT // 16)
        def _zd(i):
            den_b[pl.ds(i * 16, 16)] = zero16

        base = s * RPT
        for i in range(RPT // KB):
            pltpu.sync_copy(rows_v, acc_s.at[pl.ds(base + i * KB, KB)])
        for hd in range(nh):
            pltpu.sync_copy(den_b, den_sh[hd].at[pl.ds(base, RPT)])
        plsc.subcore_barrier()

        @pl.loop(0, nb)
        def _batch(b):
            for i in range(KB // 16):
                sv = src_v[b, pl.ds(i * 16, 16)] * stride
                dv = dst_v[b, pl.ds(i * 16, 16)] * stride
                for hd in range(nh):
                    gidx[2 * hd][pl.ds(i * 16, 16)] = sv + (2 * hd)
                    gidx[2 * hd + 1][pl.ds(i * 16, 16)] = dv + (2 * hd + 1)
            rcp = pltpu.async_copy(h_hbm.at[src_v.at[b]], rows_v, sem)
            cps = [pltpu.async_copy(sc_hbm.at[gidx[k]], sbufs[k], sem)
                   for k in range(2 * nh)]
            for cp in cps:
                cp.wait()
            for i in range(KB // 16):
                sl = pl.ds(i * 16, 16)
                for hd in range(nh):
                    e = sbufs[2 * hd][sl] + sbufs[2 * hd + 1][sl]
                    e = jnp.where(e > 0, e, 0.2 * e)
                    ex_bufs[hd][sl] = jnp.exp(e)
            for hd in range(nh):
                pltpu.sync_copy(ex_bufs[hd], den_sh[hd].at[dst_v.at[b]],
                                add=True)

            rcp.wait()

            @pl.loop(0, KB, unroll=4)
            def _scale(j):
                jv = jnp.zeros((16,), jnp.int32) + j
                for hd in range(nh):
                    exj = plsc.load_gather(ex_bufs[hd], [jv])
                    for cc in range(hd * CH, (hd + 1) * CH):
                        rows_v[j, pl.ds(cc * 16, 16)] = (
                            rows_v[j, pl.ds(cc * 16, 16)] * exj)

            pltpu.sync_copy(rows_v, acc_s.at[dst_v.at[b]], add=True)

        plsc.subcore_barrier()

        for i in range(RPT // KB):
            pltpu.sync_copy(acc_s.at[pl.ds(base + i * KB, KB)], rows_v)
            pltpu.sync_copy(rows_v, rows_out.at[c, pl.ds(base + i * KB, KB)])
        for hd in range(nh):
            pltpu.sync_copy(den_sh[hd].at[pl.ds(base, RPT)], den_b)
            pltpu.sync_copy(den_b, den_outs[hd].at[c, pl.ds(base, RPT)])

    return gat


def _proj1_body(x_ref, w1_ref, wm1_ref, a1_ref, h_ref, xw_ref, sc_ref):
    xb = x_ref[...]
    h = jnp.dot(xb, w1_ref[...], preferred_element_type=jnp.float32)
    h_ref[...] = h
    xw_ref[...] = jnp.dot(xb, wm1_ref[...], preferred_element_type=jnp.float32)
    sc_ref[...] = jnp.dot(h, a1_ref[...], preferred_element_type=jnp.float32)


def _gcn1_body(adj_ref, xw_ref, wm23_ref, bm1_ref, xlm_ref, t23_ref):
    acc = jnp.dot(adj_ref[...], xw_ref[...], preferred_element_type=jnp.float32)
    xlm = jnp.maximum(acc + bm1_ref[...], 0.0)
    xlm_ref[...] = xlm
    t23_ref[...] = jnp.dot(xlm, wm23_ref[...],
                           preferred_element_type=jnp.float32)


def _gcn2_body(adj_ref, t23_ref, bm2_ref, bm3_ref, mu_ref, lv_ref):
    acc = jnp.dot(adj_ref[...], t23_ref[...], preferred_element_type=jnp.float32)
    mu_ref[...] = acc[:, :H2] + bm2_ref[...]
    lv_ref[...] = acc[:, H2:] + bm3_ref[...]


def _fin1_body(rp_ref, dp_ref, b1_ref, w23_ref, a23_ref,
               xl_ref, h23_ref, sc_ref):
    rp = rp_ref[...]
    dp = dp_ref[...]
    den = dp[0, 0] + dp[0, 1]
    xl = jnp.maximum((rp[0] + rp[1]) / (den[:, None] + 1e-16) + b1_ref[...],
                     0.0)
    xl_ref[...] = xl
    h23 = jnp.dot(xl, w23_ref[...], preferred_element_type=jnp.float32)
    h23_ref[...] = h23
    sc_ref[...] = jnp.dot(h23, a23_ref[...], preferred_element_type=jnp.float32)


def _fin23_body(rp_ref, dA_ref, dB_ref, b2_ref, b3_ref, mu_ref, lv_ref):
    rp = rp_ref[...]
    r = rp[0] + rp[1]
    dA = dA_ref[...]
    dB = dB_ref[...]
    denA = dA[0, 0] + dA[0, 1]
    denB = dB[0, 0] + dB[0, 1]
    mu_ref[...] = r[:, :H2] / (denA[:, None] + 1e-16) + b2_ref[...]
    lv_ref[...] = r[:, H2:] / (denB[:, None] + 1e-16) + b3_ref[...]


def _full(shape):
    return pl.BlockSpec(shape, lambda i: tuple(0 for _ in shape))


def kernel(x, edge_index, adj, W1, a1s, a1d, b1, W2, a2s, a2d, b2,
           W3, a3s, a3d, b3, Wm1, bm1, Wm2, bm2, Wm3, bm3):
    n = x.shape[0]
    d_in = x.shape[1]
    h1d = W1.shape[1]
    e = edge_index.shape[1]
    f32 = jnp.float32

    loops = jnp.arange(n, dtype=edge_index.dtype)
    src = jnp.concatenate([edge_index[0], loops])
    dst = jnp.concatenate([edge_index[1], loops])
    ne = e + n
    nb = -(-ne // (NW * KB))
    pad = NW * nb * KB - ne
    src = jnp.concatenate([src, jnp.zeros((pad,), src.dtype)])
    dst = jnp.concatenate([dst, jnp.full((pad,), n, dst.dtype)])
    src = src.reshape(NW, nb, KB)
    dst = dst.reshape(NW, nb, KB)

    A1 = jnp.stack([a1s, a1d], axis=1)
    z = jnp.zeros_like(a2s)
    A23 = jnp.stack([jnp.concatenate([a2s, z]),
                     jnp.concatenate([a2d, z]),
                     jnp.concatenate([z, a3s]),
                     jnp.concatenate([z, a3d])], axis=1)
    W23 = jnp.concatenate([W2, W3], axis=1)
    Wm23 = jnp.concatenate([Wm2, Wm3], axis=1)
    b1r = b1.reshape(1, h1d)
    bm1r = bm1.reshape(1, h1d)
    b2r = b2.reshape(1, H2)
    b3r = b3.reshape(1, H2)
    bm2r = bm2.reshape(1, H2)
    bm3r = bm3.reshape(1, H2)

    BN = 1000
    gn = n // BN
    BR = 200
    gr = n // BR

    h1, xw, sc1 = pl.pallas_call(
        _proj1_body,
        grid=(gn,),
        in_specs=[pl.BlockSpec((BN, d_in), lambda i: (i, 0)),
                  _full((d_in, h1d)), _full((d_in, h1d)), _full((d_in, 2))],
        out_specs=[pl.BlockSpec((BN, h1d), lambda i: (i, 0)),
                   pl.BlockSpec((BN, h1d), lambda i: (i, 0)),
                   pl.BlockSpec((BN, 2), lambda i: (i, 0))],
        out_shape=[jax.ShapeDtypeStruct((n, h1d), f32),
                   jax.ShapeDtypeStruct((n, h1d), f32),
                   jax.ShapeDtypeStruct((n, 2), f32)],
    )(x, W1, Wm1, A1)

    sc1p = jnp.pad(sc1, ((0, NSEG_PAD - n), (0, 0))).reshape(-1)
    rows1, den1 = _make_gat_sc(1, n, nb)(h1, src, dst, sc1p)

    xlm, t23 = pl.pallas_call(
        _gcn1_body,
        grid=(gr,),
        in_specs=[pl.BlockSpec((BR, n), lambda i: (i, 0)),
                  _full((n, h1d)), _full((h1d, h1d)), _full((1, h1d))],
        out_specs=[pl.BlockSpec((BR, h1d), lambda i: (i, 0)),
                   pl.BlockSpec((BR, h1d), lambda i: (i, 0))],
        out_shape=[jax.ShapeDtypeStruct((n, h1d), f32),
                   jax.ShapeDtypeStruct((n, h1d), f32)],
    )(adj, xw, Wm23, bm1r)

    den1r = den1[:, :n].reshape(NC, gn, BN).transpose(1, 0, 2)
    xl, h23, sc23 = pl.pallas_call(
        _fin1_body,
        grid=(gn,),
        in_specs=[pl.BlockSpec((NC, BN, h1d), lambda i: (0, i, 0)),
                  pl.BlockSpec((1, NC, BN), lambda i: (i, 0, 0)),
                  _full((1, h1d)), _full((h1d, h1d)), _full((h1d, 4))],
        out_specs=[pl.BlockSpec((BN, h1d), lambda i: (i, 0)),
                   pl.BlockSpec((BN, h1d), lambda i: (i, 0)),
                   pl.BlockSpec((BN, 4), lambda i: (i, 0))],
        out_shape=[jax.ShapeDtypeStruct((n, h1d), f32),
                   jax.ShapeDtypeStruct((n, h1d), f32),
                   jax.ShapeDtypeStruct((n, 4), f32)],
    )(rows1, den1r, b1r, W23, A23)

    sc23p = jnp.pad(sc23, ((0, NSEG_PAD - n), (0, 0))).reshape(-1)
    rows23, den2p, den3p = _make_gat_sc(2, n, nb)(h23, src, dst, sc23p)

    mu_m, logvar_m = pl.pallas_call(
        _gcn2_body,
        grid=(gr,),
        in_specs=[pl.BlockSpec((BR, n), lambda i: (i, 0)),
                  _full((n, h1d)), _full((1, H2)), _full((1, H2))],
        out_specs=[pl.BlockSpec((BR, H2), lambda i: (i, 0)),
                   pl.BlockSpec((BR, H2), lambda i: (i, 0))],
        out_shape=[jax.ShapeDtypeStruct((n, H2), f32),
                   jax.ShapeDtypeStruct((n, H2), f32)],
    )(adj, t23, bm2r, bm3r)

    den2r = den2p[:, :n].reshape(NC, gn, BN).transpose(1, 0, 2)
    den3r = den3p[:, :n].reshape(NC, gn, BN).transpose(1, 0, 2)
    mu, logvar = pl.pallas_call(
        _fin23_body,
        grid=(gn,),
        in_specs=[pl.BlockSpec((NC, BN, h1d), lambda i: (0, i, 0)),
                  pl.BlockSpec((1, NC, BN), lambda i: (i, 0, 0)),
                  pl.BlockSpec((1, NC, BN), lambda i: (i, 0, 0)),
                  _full((1, H2)), _full((1, H2))],
        out_specs=[pl.BlockSpec((BN, H2), lambda i: (i, 0)),
                   pl.BlockSpec((BN, H2), lambda i: (i, 0))],
        out_shape=[jax.ShapeDtypeStruct((n, H2), f32),
                   jax.ShapeDtypeStruct((n, H2), f32)],
    )(rows23, den2r, den3r, b2r, b3r)

    return (xl, mu, mu, logvar, xlm, mu_m, mu_m, logvar_m)


# speedup vs baseline: 33.5393x; 1.0280x over previous
import functools

import jax
import jax.numpy as jnp
from jax import lax
from jax.experimental import pallas as pl
from jax.experimental.pallas import tpu as pltpu
from jax.experimental.pallas import tpu_sc as plsc

H2 = 64
NC, NS = 2, 16
NW = NC * NS
KB = 128
NSEG_PAD = 10240
RPT = NSEG_PAD // NS


def _make_gat_sc(nh, n_nodes, nb):
    D = 128
    CH = 8 // nh
    mesh = plsc.VectorSubcoreMesh(core_axis_name="c", subcore_axis_name="s")
    out_type = [jax.ShapeDtypeStruct((NC, NSEG_PAD, D), jnp.float32)] + [
        jax.ShapeDtypeStruct((NC, NSEG_PAD), jnp.float32) for _ in range(nh)
    ]
    scratch = (
        [
            pltpu.VMEM((nb, KB), jnp.int32),
            pltpu.VMEM((nb, KB), jnp.int32),
        ]
        + [pltpu.VMEM((KB,), jnp.int32) for _ in range(2 * nh)]
        + [pltpu.VMEM((KB,), jnp.float32) for _ in range(2 * nh)]
        + [pltpu.VMEM((KB,), jnp.float32) for _ in range(nh)]
        + [
            pltpu.VMEM((KB, D), jnp.float32),
            pltpu.VMEM((RPT,), jnp.float32),
            pltpu.VMEM_SHARED((NSEG_PAD, D), jnp.float32),
        ]
        + [pltpu.VMEM_SHARED((NSEG_PAD,), jnp.float32) for _ in range(nh)]
        + [pltpu.SemaphoreType.DMA, pltpu.SemaphoreType.DMA]
    )

    @functools.partial(
        pl.kernel, out_type=out_type, mesh=mesh, scratch_types=scratch,
        compiler_params=pltpu.CompilerParams(needs_layout_passes=False))
    def gat(h_hbm, src_hbm, dst_hbm, sc_hbm, rows_out, *rest):
        den_outs = rest[:nh]
        r2 = rest[nh:]
        src_v, dst_v = r2[0], r2[1]
        gidx = r2[2:2 + 2 * nh]
        sbufs = r2[2 + 2 * nh:2 + 4 * nh]
        ex_bufs = r2[2 + 4 * nh:2 + 5 * nh]
        rows_v, den_b, acc_s = r2[2 + 5 * nh:5 + 5 * nh]
        den_sh = r2[5 + 5 * nh:5 + 6 * nh]
        sem, sem2 = r2[-2], r2[-1]
        stride = 2 * nh

        c = lax.axis_index("c")
        s = lax.axis_index("s")
        w = c * NS + s
        zero16 = jnp.zeros((16,), jnp.float32)

        pltpu.sync_copy(src_hbm.at[w], src_v)
        pltpu.sync_copy(dst_hbm.at[w], dst_v)

        @pl.loop(0, KB)
        def _zr(r):
            for cc in range(8):
                rows_v[r, pl.ds(cc * 16, 16)] = zero16

        @pl.loop(0, RPT // 16)
        def _zd(i):
            den_b[pl.ds(i * 16, 16)] = zero16

        base = s * RPT
        for i in range(RPT // KB):
            pltpu.sync_copy(rows_v, acc_s.at[pl.ds(base + i * KB, KB)])
        for hd in range(nh):
            pltpu.sync_copy(den_b, den_sh[hd].at[pl.ds(base, RPT)])
        plsc.subcore_barrier()

        @pl.loop(0, nb)
        def _batch(b):
            for i in range(KB // 16):
                sv = src_v[b, pl.ds(i * 16, 16)] * stride
                dv = dst_v[b, pl.ds(i * 16, 16)] * stride
                for hd in range(nh):
                    gidx[2 * hd][pl.ds(i * 16, 16)] = sv + (2 * hd)
                    gidx[2 * hd + 1][pl.ds(i * 16, 16)] = dv + (2 * hd + 1)
            rcp = pltpu.async_copy(h_hbm.at[src_v.at[b]], rows_v, sem2)
            cps = [pltpu.async_copy(sc_hbm.at[gidx[k]], sbufs[k], sem)
                   for k in range(2 * nh)]
            for cp in cps:
                cp.wait()
            for i in range(KB // 16):
                sl = pl.ds(i * 16, 16)
                for hd in range(nh):
                    e = sbufs[2 * hd][sl] + sbufs[2 * hd + 1][sl]
                    e = jnp.where(e > 0, e, 0.2 * e)
                    ex_bufs[hd][sl] = jnp.exp(e)
            for hd in range(nh):
                pltpu.sync_copy(ex_bufs[hd], den_sh[hd].at[dst_v.at[b]],
                                add=True)

            rcp.wait()

            @pl.loop(0, KB, unroll=4)
            def _scale(j):
                jv = jnp.zeros((16,), jnp.int32) + j
                for hd in range(nh):
                    exj = plsc.load_gather(ex_bufs[hd], [jv])
                    for cc in range(hd * CH, (hd + 1) * CH):
                        rows_v[j, pl.ds(cc * 16, 16)] = (
                            rows_v[j, pl.ds(cc * 16, 16)] * exj)

            pltpu.sync_copy(rows_v, acc_s.at[dst_v.at[b]], add=True)

        plsc.subcore_barrier()

        for i in range(RPT // KB):
            pltpu.sync_copy(acc_s.at[pl.ds(base + i * KB, KB)], rows_v)
            pltpu.sync_copy(rows_v, rows_out.at[c, pl.ds(base + i * KB, KB)])
        for hd in range(nh):
            pltpu.sync_copy(den_sh[hd].at[pl.ds(base, RPT)], den_b)
            pltpu.sync_copy(den_b, den_outs[hd].at[c, pl.ds(base, RPT)])

    return gat


def _proj1_body(x_ref, w1_ref, wm1_ref, a1_ref, h_ref, xw_ref, sc_ref):
    xb = x_ref[...]
    h = jnp.dot(xb, w1_ref[...], preferred_element_type=jnp.float32)
    h_ref[...] = h
    xw_ref[...] = jnp.dot(xb, wm1_ref[...], preferred_element_type=jnp.float32)
    sc_ref[...] = jnp.dot(h, a1_ref[...], preferred_element_type=jnp.float32)


def _gcn1_body(adj_ref, xw_ref, wm23_ref, bm1_ref, xlm_ref, t23_ref):
    acc = jnp.dot(adj_ref[...], xw_ref[...], preferred_element_type=jnp.float32)
    xlm = jnp.maximum(acc + bm1_ref[...], 0.0)
    xlm_ref[...] = xlm
    t23_ref[...] = jnp.dot(xlm, wm23_ref[...],
                           preferred_element_type=jnp.float32)


def _gcn2_body(adj_ref, t23_ref, bm2_ref, bm3_ref, mu_ref, lv_ref):
    acc = jnp.dot(adj_ref[...], t23_ref[...], preferred_element_type=jnp.float32)
    mu_ref[...] = acc[:, :H2] + bm2_ref[...]
    lv_ref[...] = acc[:, H2:] + bm3_ref[...]


def _fin1_body(rp_ref, dp_ref, b1_ref, w23_ref, a23_ref,
               xl_ref, h23_ref, sc_ref):
    rp = rp_ref[...]
    dp = dp_ref[...]
    den = dp[0, 0] + dp[0, 1]
    xl = jnp.maximum((rp[0] + rp[1]) / (den[:, None] + 1e-16) + b1_ref[...],
                     0.0)
    xl_ref[...] = xl
    h23 = jnp.dot(xl, w23_ref[...], preferred_element_type=jnp.float32)
    h23_ref[...] = h23
    sc_ref[...] = jnp.dot(h23, a23_ref[...], preferred_element_type=jnp.float32)


def _fin23_body(rp_ref, dA_ref, dB_ref, b2_ref, b3_ref, mu_ref, lv_ref):
    rp = rp_ref[...]
    r = rp[0] + rp[1]
    dA = dA_ref[...]
    dB = dB_ref[...]
    denA = dA[0, 0] + dA[0, 1]
    denB = dB[0, 0] + dB[0, 1]
    mu_ref[...] = r[:, :H2] / (denA[:, None] + 1e-16) + b2_ref[...]
    lv_ref[...] = r[:, H2:] / (denB[:, None] + 1e-16) + b3_ref[...]


def _full(shape):
    return pl.BlockSpec(shape, lambda i: tuple(0 for _ in shape))


def kernel(x, edge_index, adj, W1, a1s, a1d, b1, W2, a2s, a2d, b2,
           W3, a3s, a3d, b3, Wm1, bm1, Wm2, bm2, Wm3, bm3):
    n = x.shape[0]
    d_in = x.shape[1]
    h1d = W1.shape[1]
    e = edge_index.shape[1]
    f32 = jnp.float32

    loops = jnp.arange(n, dtype=edge_index.dtype)
    src = jnp.concatenate([edge_index[0], loops])
    dst = jnp.concatenate([edge_index[1], loops])
    ne = e + n
    nb = -(-ne // (NW * KB))
    pad = NW * nb * KB - ne
    src = jnp.concatenate([src, jnp.zeros((pad,), src.dtype)])
    dst = jnp.concatenate([dst, jnp.full((pad,), n, dst.dtype)])
    src = src.reshape(NW, nb, KB)
    dst = dst.reshape(NW, nb, KB)

    A1 = jnp.stack([a1s, a1d], axis=1)
    z = jnp.zeros_like(a2s)
    A23 = jnp.stack([jnp.concatenate([a2s, z]),
                     jnp.concatenate([a2d, z]),
                     jnp.concatenate([z, a3s]),
                     jnp.concatenate([z, a3d])], axis=1)
    W23 = jnp.concatenate([W2, W3], axis=1)
    Wm23 = jnp.concatenate([Wm2, Wm3], axis=1)
    b1r = b1.reshape(1, h1d)
    bm1r = bm1.reshape(1, h1d)
    b2r = b2.reshape(1, H2)
    b3r = b3.reshape(1, H2)
    bm2r = bm2.reshape(1, H2)
    bm3r = bm3.reshape(1, H2)

    BN = 1000
    gn = n // BN
    BR = 200
    gr = n // BR

    h1, xw, sc1 = pl.pallas_call(
        _proj1_body,
        grid=(gn,),
        in_specs=[pl.BlockSpec((BN, d_in), lambda i: (i, 0)),
                  _full((d_in, h1d)), _full((d_in, h1d)), _full((d_in, 2))],
        out_specs=[pl.BlockSpec((BN, h1d), lambda i: (i, 0)),
                   pl.BlockSpec((BN, h1d), lambda i: (i, 0)),
                   pl.BlockSpec((BN, 2), lambda i: (i, 0))],
        out_shape=[jax.ShapeDtypeStruct((n, h1d), f32),
                   jax.ShapeDtypeStruct((n, h1d), f32),
                   jax.ShapeDtypeStruct((n, 2), f32)],
    )(x, W1, Wm1, A1)

    sc1p = jnp.pad(sc1, ((0, NSEG_PAD - n), (0, 0))).reshape(-1)
    rows1, den1 = _make_gat_sc(1, n, nb)(h1, src, dst, sc1p)

    xlm, t23 = pl.pallas_call(
        _gcn1_body,
        grid=(gr,),
        in_specs=[pl.BlockSpec((BR, n), lambda i: (i, 0)),
                  _full((n, h1d)), _full((h1d, h1d)), _full((1, h1d))],
        out_specs=[pl.BlockSpec((BR, h1d), lambda i: (i, 0)),
                   pl.BlockSpec((BR, h1d), lambda i: (i, 0))],
        out_shape=[jax.ShapeDtypeStruct((n, h1d), f32),
                   jax.ShapeDtypeStruct((n, h1d), f32)],
    )(adj, xw, Wm23, bm1r)

    den1r = den1[:, :n].reshape(NC, gn, BN).transpose(1, 0, 2)
    xl, h23, sc23 = pl.pallas_call(
        _fin1_body,
        grid=(gn,),
        in_specs=[pl.BlockSpec((NC, BN, h1d), lambda i: (0, i, 0)),
                  pl.BlockSpec((1, NC, BN), lambda i: (i, 0, 0)),
                  _full((1, h1d)), _full((h1d, h1d)), _full((h1d, 4))],
        out_specs=[pl.BlockSpec((BN, h1d), lambda i: (i, 0)),
                   pl.BlockSpec((BN, h1d), lambda i: (i, 0)),
                   pl.BlockSpec((BN, 4), lambda i: (i, 0))],
        out_shape=[jax.ShapeDtypeStruct((n, h1d), f32),
                   jax.ShapeDtypeStruct((n, h1d), f32),
                   jax.ShapeDtypeStruct((n, 4), f32)],
    )(rows1, den1r, b1r, W23, A23)

    sc23p = jnp.pad(sc23, ((0, NSEG_PAD - n), (0, 0))).reshape(-1)
    rows23, den2p, den3p = _make_gat_sc(2, n, nb)(h23, src, dst, sc23p)

    mu_m, logvar_m = pl.pallas_call(
        _gcn2_body,
        grid=(gr,),
        in_specs=[pl.BlockSpec((BR, n), lambda i: (i, 0)),
                  _full((n, h1d)), _full((1, H2)), _full((1, H2))],
        out_specs=[pl.BlockSpec((BR, H2), lambda i: (i, 0)),
                   pl.BlockSpec((BR, H2), lambda i: (i, 0))],
        out_shape=[jax.ShapeDtypeStruct((n, H2), f32),
                   jax.ShapeDtypeStruct((n, H2), f32)],
    )(adj, t23, bm2r, bm3r)

    den2r = den2p[:, :n].reshape(NC, gn, BN).transpose(1, 0, 2)
    den3r = den3p[:, :n].reshape(NC, gn, BN).transpose(1, 0, 2)
    mu, logvar = pl.pallas_call(
        _fin23_body,
        grid=(gn,),
        in_specs=[pl.BlockSpec((NC, BN, h1d), lambda i: (0, i, 0)),
                  pl.BlockSpec((1, NC, BN), lambda i: (i, 0, 0)),
                  pl.BlockSpec((1, NC, BN), lambda i: (i, 0, 0)),
                  _full((1, H2)), _full((1, H2))],
        out_specs=[pl.BlockSpec((BN, H2), lambda i: (i, 0)),
                   pl.BlockSpec((BN, H2), lambda i: (i, 0))],
        out_shape=[jax.ShapeDtypeStruct((n, H2), f32),
                   jax.ShapeDtypeStruct((n, H2), f32)],
    )(rows23, den2r, den3r, b2r, b3r)

    return (xl, mu, mu, logvar, xlm, mu_m, mu_m, logvar_m)
